# Initial kernel scaffold; baseline (speedup 1.0000x reference)
import jax
import jax.numpy as jnp
from jax.experimental import pallas as pl


def kernel(x, edge_index, hyperedge_index, W_g1, b_g1, W_g2, b_g2, W_h1, b_h1, W_h2, b_h2, ln1_g, ln1_b, ln2_g, ln2_b, Wp1, bp1, Wp2, bp2):
    raise NotImplementedError("write your pallas kernel here")


# speedup vs baseline: 10.6609x; 10.6609x over previous
import functools

import jax
import jax.numpy as jnp
from jax import lax
from jax.experimental import pallas as pl
from jax.experimental.pallas import tpu as pltpu
from jax.experimental.pallas import tpu_sc as plsc

N = 10000
E = 320000
D = 128

NC = 2
NS = 16
NW = NC * NS
EPW = E // NW
K = 80
CPW = EPW // K
K2 = 128
CP2 = 80
EPW2 = K2 * CP2
E2 = NW * EPW2
NPAD = 10112
RPS = NPAD // NS


@functools.lru_cache(maxsize=None)
def _build_sc_kernels():
    mesh = plsc.VectorSubcoreMesh(
        core_axis_name="c", subcore_axis_name="s",
        num_cores=NC, num_subcores=NS)

    @functools.partial(
        pl.kernel,
        out_type=jax.ShapeDtypeStruct((NC * 3 * NPAD,), jnp.float32),
        mesh=mesh,
        scratch_types=[
            pltpu.VMEM((K,), jnp.float32),
            [pltpu.VMEM((K,), jnp.int32) for _ in range(6)],
            pltpu.VMEM((RPS,), jnp.float32),
            pltpu.SemaphoreType.DMA,
            pltpu.VMEM_SHARED((NPAD,), jnp.float32),
            pltpu.VMEM_SHARED((NPAD,), jnp.float32),
            pltpu.VMEM_SHARED((NPAD,), jnp.float32),
        ],
    )
    def sc_degrees(dst_g, node, hedge, ones_hbm, z1_hbm, out,
                   ones_v, idx, bounce, sems, acc0, acc1, acc2):
        cid = lax.axis_index("c")
        sid = lax.axis_index("s")
        wid = cid * NS + sid
        row0 = sid * RPS
        pltpu.sync_copy(z1_hbm.at[pl.ds(row0, RPS)], bounce)
        for acc in (acc0, acc1, acc2):
            pltpu.sync_copy(bounce, acc.at[pl.ds(row0, RPS)])
        pltpu.sync_copy(ones_hbm, ones_v)
        plsc.subcore_barrier()

        def drain3(j):
            for _ in range(3):
                pltpu.make_async_copy(z1_hbm.at[pl.ds(0, K)], ones_v,
                                      sems).wait()

        def chunk(j, carry):
            for u in range(2):
                i = 2 * j + u
                o = 3 * u
                base = pl.multiple_of(wid * EPW + i * K, 8)

                @pl.when(i >= 2)
                def _():
                    drain3(i - 2)
                pltpu.sync_copy(dst_g.at[pl.ds(base, K)], idx[o])
                pltpu.sync_copy(node.at[pl.ds(base, K)], idx[o + 1])
                pltpu.sync_copy(hedge.at[pl.ds(base, K)], idx[o + 2])
                pltpu.async_copy(ones_v, acc0.at[idx[o]], sems, add=True)
                pltpu.async_copy(ones_v, acc1.at[idx[o + 1]], sems, add=True)
                pltpu.async_copy(ones_v, acc2.at[idx[o + 2]], sems, add=True)
            return carry

        lax.fori_loop(0, CPW // 2, chunk, 0)
        drain3(CPW - 3)
        base = pl.multiple_of(wid * EPW + (CPW - 1) * K, 8)
        pltpu.sync_copy(dst_g.at[pl.ds(base, K)], idx[0])
        pltpu.sync_copy(node.at[pl.ds(base, K)], idx[1])
        pltpu.sync_copy(hedge.at[pl.ds(base, K)], idx[2])
        pltpu.async_copy(ones_v, acc0.at[idx[0]], sems, add=True)
        pltpu.async_copy(ones_v, acc1.at[idx[1]], sems, add=True)
        pltpu.async_copy(ones_v, acc2.at[idx[2]], sems, add=True)
        drain3(CPW - 2)
        drain3(CPW - 1)
        plsc.subcore_barrier()
        for a, acc in enumerate((acc0, acc1, acc2)):
            pltpu.sync_copy(acc.at[pl.ds(row0, RPS)], bounce)
            dst_off = pl.multiple_of(cid * 3 * NPAD + a * NPAD + row0, 8)
            pltpu.sync_copy(bounce, out.at[pl.ds(dst_off, RPS)])

    @functools.partial(
        pl.kernel,
        out_type=jax.ShapeDtypeStruct((NC, NPAD, D), jnp.float32),
        mesh=mesh,
        scratch_types=[
            pltpu.VMEM((K,), jnp.int32),
            pltpu.VMEM((K,), jnp.int32),
            pltpu.VMEM((K,), jnp.int32),
            pltpu.VMEM((K,), jnp.int32),
            pltpu.VMEM((K, D), jnp.float32),
            pltpu.VMEM((K, D), jnp.float32),
            pltpu.SemaphoreType.DMA,
            pltpu.SemaphoreType.DMA,
            pltpu.VMEM_SHARED((NPAD, D), jnp.float32),
        ],
    )
    def sc_segsum(table, src, dst, z128_hbm, out,
                  s0, d0, s1, d1, rows0, rows1, semg, sems, acc):
        cid = lax.axis_index("c")
        sid = lax.axis_index("s")
        wid = cid * NS + sid
        row0 = sid * RPS
        pltpu.sync_copy(z128_hbm.at[pl.ds(row0, RPS)], acc.at[pl.ds(row0, RPS)])
        plsc.subcore_barrier()

        def drain_scat(buf):
            pltpu.make_async_copy(z128_hbm.at[pl.ds(0, K)], buf, sems).wait()

        def load_idx(i, sv, dv):
            base = pl.multiple_of(wid * EPW + i * K, 8)
            pltpu.sync_copy(src.at[pl.ds(base, K)], sv)
            pltpu.sync_copy(dst.at[pl.ds(base, K)], dv)

        load_idx(0, s0, d0)

        def pair(j, carry):
            i0 = 2 * j
            g0 = pltpu.async_copy(table.at[s0], rows0, semg)

            @pl.when(j > 0)
            def _():
                drain_scat(rows1)
            load_idx(i0 + 1, s1, d1)
            g0.wait()
            pltpu.async_copy(rows0, acc.at[d0], sems, add=True)

            g1 = pltpu.async_copy(table.at[s1], rows1, semg)
            drain_scat(rows0)
            load_idx(i0 + 2, s0, d0)
            g1.wait()
            pltpu.async_copy(rows1, acc.at[d1], sems, add=True)
            return carry

        lax.fori_loop(0, CPW // 2, pair, 0)
        pltpu.async_copy(table.at[s0], rows0, semg).wait()
        drain_scat(rows1)
        pltpu.sync_copy(rows0, acc.at[d0], add=True)
        plsc.subcore_barrier()
        pltpu.sync_copy(acc.at[pl.ds(row0, RPS)], out.at[cid, pl.ds(row0, RPS)])

    return sc_degrees, sc_segsum


def _sc_degrees(dst_g, node, hedge, ones1, z1):
    flat = _build_sc_kernels()[0](dst_g, node, hedge, ones1, z1)
    return flat.reshape(NC, 3, NPAD, 1)


def _sc_segsum(table, src, dst, z128):
    return _build_sc_kernels()[1](table, src, dst, z128)


RB = 2000


def _scalars_body(degp_ref, out_ref):
    d = degp_ref[0, 0] + degp_ref[1, 0] + 1.0
    out_ref[0] = lax.rsqrt(d)
    dn = degp_ref[0, 1] + degp_ref[1, 1]
    out_ref[1] = jnp.where(dn > 0, 1.0 / dn, 0.0)
    be = degp_ref[0, 2] + degp_ref[1, 2]
    out_ref[2] = jnp.where(be > 0, 1.0 / be, 0.0)


def _tc_scalars(degp):
    return pl.pallas_call(
        _scalars_body,
        out_shape=jax.ShapeDtypeStruct((3, NPAD, 1), jnp.float32),
    )(degp)


def _xw_body(x_ref, wg_ref, wh_ref, dis_ref, og_ref, oh_ref):
    xb = x_ref[...]
    og_ref[...] = jnp.dot(xb, wg_ref[...],
                          preferred_element_type=jnp.float32) * dis_ref[...]
    oh_ref[...] = jnp.dot(xb, wh_ref[...], preferred_element_type=jnp.float32)


def _tc_xw(x, Wg, Wh, dis):
    return pl.pallas_call(
        _xw_body,
        grid=(N // RB,),
        in_specs=[
            pl.BlockSpec((RB, D), lambda i: (i, 0)),
            pl.BlockSpec((D, D), lambda i: (0, 0)),
            pl.BlockSpec((D, D), lambda i: (0, 0)),
            pl.BlockSpec((RB, 1), lambda i: (i, 0)),
        ],
        out_specs=[
            pl.BlockSpec((RB, D), lambda i: (i, 0)),
            pl.BlockSpec((RB, D), lambda i: (i, 0)),
        ],
        out_shape=[
            jax.ShapeDtypeStruct((N, D), jnp.float32),
            jax.ShapeDtypeStruct((N, D), jnp.float32),
        ],
    )(x, Wg, Wh, dis)


def _rowscale_body(acc_ref, s_ref, o_ref):
    o_ref[...] = (acc_ref[0] + acc_ref[1]) * s_ref[...]


def _tc_rowscale(acc, s):
    return pl.pallas_call(
        _rowscale_body,
        grid=(N // RB,),
        in_specs=[
            pl.BlockSpec((NC, RB, D), lambda i: (0, i, 0)),
            pl.BlockSpec((RB, 1), lambda i: (i, 0)),
        ],
        out_specs=pl.BlockSpec((RB, D), lambda i: (i, 0)),
        out_shape=jax.ShapeDtypeStruct((N, D), jnp.float32),
    )(acc, s)


def _ln(x, g, b):
    mu = jnp.mean(x, axis=-1, keepdims=True)
    var = jnp.mean((x - mu) ** 2, axis=-1, keepdims=True)
    return (x - mu) / jnp.sqrt(var + 1e-5) * g + b


def _make_post(with_extra, with_next, scale_next):
    def body(*refs):
        i = 0
        acc_ref = refs[i]; i += 1
        extra_ref = None
        if with_extra:
            extra_ref = refs[i]; i += 1
        s_ref = refs[i]; i += 1
        b_ref = refs[i]; i += 1
        g_ref = refs[i]; i += 1
        lb_ref = refs[i]; i += 1
        wn_ref = None
        if with_next:
            wn_ref = refs[i]; i += 1
        o_ref = refs[i]
        t = acc_ref[0] + acc_ref[1]
        if with_extra:
            t = t + extra_ref[...]
        t = t * s_ref[...] + b_ref[...]
        t = jax.nn.relu(_ln(t, g_ref[...], lb_ref[...]))
        if with_next:
            t = jnp.dot(t, wn_ref[...], preferred_element_type=jnp.float32)
            if scale_next:
                t = t * s_ref[...]
        o_ref[...] = t

    def call(acc, extra, s, bias, ln_g, ln_b, Wn):
        args = [acc]
        specs = [pl.BlockSpec((NC, RB, D), lambda i: (0, i, 0))]
        if with_extra:
            args.append(extra)
            specs.append(pl.BlockSpec((RB, D), lambda i: (i, 0)))
        args += [s, bias.reshape(1, D), ln_g.reshape(1, D), ln_b.reshape(1, D)]
        specs += [
            pl.BlockSpec((RB, 1), lambda i: (i, 0)),
            pl.BlockSpec((1, D), lambda i: (0, 0)),
            pl.BlockSpec((1, D), lambda i: (0, 0)),
            pl.BlockSpec((1, D), lambda i: (0, 0)),
        ]
        if with_next:
            args.append(Wn)
            specs.append(pl.BlockSpec((D, D), lambda i: (0, 0)))
        return pl.pallas_call(
            body,
            grid=(N // RB,),
            in_specs=specs,
            out_specs=pl.BlockSpec((RB, D), lambda i: (i, 0)),
            out_shape=jax.ShapeDtypeStruct((N, D), jnp.float32),
        )(*args)

    return call


_gcn_post1 = _make_post(with_extra=True, with_next=True, scale_next=True)
_gcn_post2 = _make_post(with_extra=True, with_next=False, scale_next=False)
_hgc_post1 = _make_post(with_extra=False, with_next=True, scale_next=False)
_hgc_post2 = _make_post(with_extra=False, with_next=False, scale_next=False)


def _final_body(h1_ref, h2_ref, w1_ref, b1_ref, w2_ref, b2_ref, o_ref):
    h = (h1_ref[...] + h2_ref[...]) * 0.5
    t = jax.nn.relu(jnp.dot(h, w1_ref[...],
                            preferred_element_type=jnp.float32) + b1_ref[...])
    o_ref[...] = jnp.dot(t, w2_ref[...],
                         preferred_element_type=jnp.float32) + b2_ref[...]


def _tc_final(h1, h2, Wp1, bp1, Wp2, bp2):
    return pl.pallas_call(
        _final_body,
        grid=(N // RB,),
        in_specs=[
            pl.BlockSpec((RB, D), lambda i: (i, 0)),
            pl.BlockSpec((RB, D), lambda i: (i, 0)),
            pl.BlockSpec((D, D), lambda i: (0, 0)),
            pl.BlockSpec((1, D), lambda i: (0, 0)),
            pl.BlockSpec((D, D), lambda i: (0, 0)),
            pl.BlockSpec((1, D), lambda i: (0, 0)),
        ],
        out_specs=pl.BlockSpec((RB, D), lambda i: (i, 0)),
        out_shape=jax.ShapeDtypeStruct((N, D), jnp.float32),
    )(h1, h2, Wp1, bp1.reshape(1, D), Wp2, bp2.reshape(1, D))


def kernel(x, edge_index, hyperedge_index, W_g1, b_g1, W_g2, b_g2,
           W_h1, b_h1, W_h2, b_h2, ln1_g, ln1_b, ln2_g, ln2_b,
           Wp1, bp1, Wp2, bp2):
    src_g = edge_index[0]
    dst_g = edge_index[1]
    node = hyperedge_index[0]
    hedge = hyperedge_index[1]


    ones1 = jnp.ones((K,), jnp.float32)
    z1 = jnp.zeros((NPAD,), jnp.float32)
    z128 = jnp.zeros((NPAD, D), jnp.float32)

    degp = _sc_degrees(dst_g, node, hedge, ones1, z1)
    scal = _tc_scalars(degp)
    dis, dinv, binv = scal[0], scal[1], scal[2]

    xws1, xwh1 = _tc_xw(x, W_g1, W_h1, dis)

    accg1 = _sc_segsum(xws1, src_g, dst_g, z128)
    xws2 = _gcn_post1(accg1, xws1, dis, b_g1, ln1_g, ln1_b, W_g2)
    accg2 = _sc_segsum(xws2, src_g, dst_g, z128)
    h1 = _gcn_post2(accg2, xws2, dis, b_g2, ln1_g, ln1_b, None)

    acch1 = _sc_segsum(xwh1, node, hedge, z128)
    he1 = _tc_rowscale(acch1, binv)
    accn1 = _sc_segsum(he1, hedge, node, z128)
    xwh2 = _hgc_post1(accn1, None, dinv, b_h1, ln2_g, ln2_b, W_h2)
    acch2 = _sc_segsum(xwh2, node, hedge, z128)
    he2 = _tc_rowscale(acch2, binv)
    accn2 = _sc_segsum(he2, hedge, node, z128)
    h2 = _hgc_post2(accn2, None, dinv, b_h2, ln2_g, ln2_b, None)

    return _tc_final(h1, h2, Wp1, bp1, Wp2, bp2)
